# Initial kernel scaffold; baseline (speedup 1.0000x reference)
import functools

import numpy as np
import jax
import jax.numpy as jnp
from jax import lax
from jax.experimental import pallas as pl
from jax.experimental.pallas import tpu as pltpu
from jax.experimental.pallas import tpu_sc as plsc

_R = 64
_D = 32
_K = 512
_B = 1024
_NI = 16
_N = _B * _NI
_N8 = _N * 8
_COMMIT = 0.25

_t_np = np.linspace(0.05, 1.2, _NI + 1, dtype=np.float32)
_tmid_np = 0.5 * (_t_np[:-1] + _t_np[1:])
_dt_np = (_t_np[1:] - _t_np[:-1]).astype(np.float32)
_offsets_np = np.array(
    [[i, j, k] for i in (0, 1) for j in (0, 1) for k in (0, 1)], dtype=np.int32
)

_NW = 32
_PW = _N8 // _NW
_CH = 1024
_G = 128


def _sc_gather(table, idx):
    mesh = plsc.VectorSubcoreMesh(core_axis_name="c", subcore_axis_name="s")

    @functools.partial(
        pl.kernel,
        mesh=mesh,
        out_type=jax.ShapeDtypeStruct((_N8, _D), jnp.float32),
        scratch_types=[
            pltpu.VMEM((_PW,), jnp.int32),
            pltpu.VMEM((_CH, _D), jnp.float32),
            pltpu.SemaphoreType.DMA,
        ],
    )
    def k(table_hbm, idx_hbm, out_hbm, idx_v, rows_v, sem):
        wid = lax.axis_index("s") * 2 + lax.axis_index("c")
        base = wid * _PW
        pltpu.sync_copy(idx_hbm.at[pl.ds(base, _PW)], idx_v)
        for s in range(_PW // _CH):
            cps = [
                pltpu.async_copy(
                    table_hbm.at[idx_v.at[pl.ds(s * _CH + c * _G, _G)]],
                    rows_v.at[pl.ds(c * _G, _G)],
                    sem,
                )
                for c in range(_CH // _G)
            ]
            for cp in cps:
                cp.wait()
            pltpu.sync_copy(rows_v, out_hbm.at[pl.ds(base + s * _CH, _CH)])

    return k(table, idx)


_TN = 512
_NT = _N // _TN


def _vq_body(z_ref, iw_ref, v_ref, ct_ref, c4_ref,
             interp_ref, counts_ref, loss_ref, ms_ref, ow_acc):
    t = pl.program_id(0)
    j = pl.program_id(1)
    z = z_ref[0]
    ct = ct_ref[...]
    zc = jnp.dot(z, ct, preferred_element_type=jnp.float32)
    znorm = jnp.sum(z * z, axis=1, keepdims=True)
    cn = jnp.sum(ct * ct, axis=0, keepdims=True)
    d = znorm - 2.0 * zc + cn
    dmin = jnp.min(d, axis=1, keepdims=True)
    idx = jnp.argmin(d, axis=1)
    onehot = lax.broadcasted_iota(jnp.int32, (_TN, _K), 1) == idx[:, None]
    v = v_ref[...]
    wv = iw_ref[0] * v
    ow = jnp.where(onehot, wv, 0.0)

    first = jnp.logical_and(t == 0, j == 0)

    @pl.when(first)
    def _init():
        counts_ref[...] = jnp.zeros_like(counts_ref)
        loss_ref[...] = jnp.zeros_like(loss_ref)
        ms_ref[...] = jnp.zeros_like(ms_ref)

    @pl.when(j == 0)
    def _start_tile():
        ow_acc[...] = ow
        ms_ref[...] += jnp.sum(v).reshape(1, 1)

    @pl.when(j > 0)
    def _accum():
        ow_acc[...] += ow

    @pl.when(j == 7)
    def _emit():
        interp_ref[...] = jnp.dot(
            ow_acc[...], c4_ref[...], preferred_element_type=jnp.float32
        )

    ov = jnp.where(onehot, v, 0.0)
    counts_ref[...] += jnp.sum(ov, axis=0, keepdims=True)
    loss_ref[...] += jnp.sum(v * dmin).reshape(1, 1)


def _run_vq(zarr, iw_t, vf, ct, c4):
    return pl.pallas_call(
        _vq_body,
        grid=(_NT, 8),
        in_specs=[
            pl.BlockSpec((1, _TN, _D), lambda t, j: (j, t, 0)),
            pl.BlockSpec((1, _TN, 1), lambda t, j: (j, t, 0)),
            pl.BlockSpec((_TN, 1), lambda t, j: (t, 0)),
            pl.BlockSpec((_D, _K), lambda t, j: (0, 0)),
            pl.BlockSpec((_K, 4), lambda t, j: (0, 0)),
        ],
        out_specs=[
            pl.BlockSpec((_TN, 4), lambda t, j: (t, 0)),
            pl.BlockSpec((1, _K), lambda t, j: (0, 0)),
            pl.BlockSpec((1, 1), lambda t, j: (0, 0)),
            pl.BlockSpec((1, 1), lambda t, j: (0, 0)),
        ],
        out_shape=[
            jax.ShapeDtypeStruct((_N, 4), jnp.float32),
            jax.ShapeDtypeStruct((1, _K), jnp.float32),
            jax.ShapeDtypeStruct((1, 1), jnp.float32),
            jax.ShapeDtypeStruct((1, 1), jnp.float32),
        ],
        scratch_shapes=[pltpu.VMEM((_TN, _K), jnp.float32)],
    )(zarr, iw_t, vf, ct, c4)


def _sigm(x):
    return 1.0 / (1.0 + jnp.exp(-x))


def _fin_body(t4_ref, counts_ref, loss_ref, ms_ref,
              vql_ref, perp_ref, r0_ref, r1_ref, r2_ref):
    ms8 = ms_ref[...] * 8.0
    vql_ref[...] = (1.0 + _COMMIT) * loss_ref[...] / (ms8 * _D + 1e-9)
    avg = counts_ref[...] / (ms8 + 1e-9)
    perp_ref[...] = jnp.exp(-jnp.sum(avg * jnp.log(avg + 1e-10))).reshape(1, 1)

    trans = jnp.ones((1, _B), jnp.float32)
    a0 = jnp.zeros((1, _B), jnp.float32)
    a1 = jnp.zeros((1, _B), jnp.float32)
    a2 = jnp.zeros((1, _B), jnp.float32)
    for i in range(_NI):
        blk = t4_ref[i]
        sig = jnp.maximum(blk[0:1, :], 0.0)
        alpha = 1.0 - jnp.exp(-sig * float(_dt_np[i]))
        w = alpha * trans
        a0 = a0 + w * _sigm(blk[1:2, :])
        a1 = a1 + w * _sigm(blk[2:3, :])
        a2 = a2 + w * _sigm(blk[3:4, :])
        trans = trans * (1.0 - alpha + 1e-10)
    r0_ref[...] = a0
    r1_ref[...] = a1
    r2_ref[...] = a2


def _run_fin(t4, counts, loss, ms):
    return pl.pallas_call(
        _fin_body,
        out_shape=[
            jax.ShapeDtypeStruct((1, 1), jnp.float32),
            jax.ShapeDtypeStruct((1, 1), jnp.float32),
            jax.ShapeDtypeStruct((1, _B), jnp.float32),
            jax.ShapeDtypeStruct((1, _B), jnp.float32),
            jax.ShapeDtypeStruct((1, _B), jnp.float32),
        ],
    )(t4, counts, loss, ms)


def kernel(rays_o, rays_d, data_weight, codebook):
    tm = jnp.asarray(_tmid_np)
    pts = rays_o[:, None, :] + tm[None, :, None] * rays_d[:, None, :]
    valid = jnp.all((pts >= 0.0) & (pts < 1.0), axis=-1)
    flat_pts = pts.reshape(-1, 3)
    clipped = jnp.clip(flat_pts, 0.0, 1.0 - 1e-6)
    scaled = clipped * _R
    idx0f = jnp.clip(jnp.floor(scaled), 0.0, _R - 1)
    frac = scaled - idx0f
    idx0 = idx0f.astype(jnp.int32)
    offs = jnp.asarray(_offsets_np)
    corners = idx0[:, None, :] + offs[None, :, :]
    nids = (corners[..., 0] * (_R + 1) + corners[..., 1]) * (_R + 1) + corners[..., 2]
    w = jnp.where(offs[None, :, :] == 1, frac[:, None, :], 1.0 - frac[:, None, :])
    iweights = jnp.prod(w, axis=-1)

    nids_t = nids.T.reshape(-1)
    iw_t = iweights.T.reshape(8, _N, 1)
    vf = valid.reshape(-1, 1).astype(jnp.float32)

    zflat = _sc_gather(data_weight, nids_t)
    zarr = zflat.reshape(8, _N, _D)

    ct = codebook.T
    c4 = codebook[:, 0:4]

    interp4, counts, loss, ms = _run_vq(zarr, iw_t, vf, ct, c4)

    t4 = interp4.reshape(_B, _NI, 4).transpose(1, 2, 0)
    vql, perp, r0, r1, r2 = _run_fin(t4, counts, loss, ms)
    rgb = jnp.concatenate([r0, r1, r2], axis=0).T
    return (vql[0, 0], perp[0, 0], rgb)


# speedup vs baseline: 1.6687x; 1.6687x over previous
import functools

import numpy as np
import jax
import jax.numpy as jnp
from jax import lax
from jax.experimental import pallas as pl
from jax.experimental.pallas import tpu as pltpu
from jax.experimental.pallas import tpu_sc as plsc

_R = 64
_D = 32
_K = 512
_B = 1024
_NI = 16
_N = _B * _NI
_N8 = _N * 8
_COMMIT = 0.25

_t_np = np.linspace(0.05, 1.2, _NI + 1, dtype=np.float32)
_tmid_np = 0.5 * (_t_np[:-1] + _t_np[1:])
_dt_np = (_t_np[1:] - _t_np[:-1]).astype(np.float32)
_offsets_np = np.array(
    [[i, j, k] for i in (0, 1) for j in (0, 1) for k in (0, 1)], dtype=np.int32
)

_NW = 32
_PW = _N8 // _NW
_CH = 1024
_G = 128


def _sc_gather(table, idx):
    mesh = plsc.VectorSubcoreMesh(core_axis_name="c", subcore_axis_name="s")

    @functools.partial(
        pl.kernel,
        mesh=mesh,
        out_type=jax.ShapeDtypeStruct((_N8, _D), jnp.float32),
        scratch_types=[
            pltpu.VMEM((_PW,), jnp.int32),
            pltpu.VMEM((_CH, _D), jnp.float32),
            pltpu.SemaphoreType.DMA,
        ],
        compiler_params=pltpu.CompilerParams(use_tc_tiling_on_sc=False),
    )
    def k(table_hbm, idx_hbm, out_hbm, idx_v, rows_v, sem):
        wid = lax.axis_index("s") * 2 + lax.axis_index("c")
        base = wid * _PW
        pltpu.sync_copy(idx_hbm.at[pl.ds(base, _PW)], idx_v)
        for s in range(_PW // _CH):
            cps = [
                pltpu.async_copy(
                    table_hbm.at[idx_v.at[pl.ds(s * _CH + c * _G, _G)]],
                    rows_v.at[pl.ds(c * _G, _G)],
                    sem,
                )
                for c in range(_CH // _G)
            ]
            for cp in cps:
                cp.wait()
            pltpu.sync_copy(rows_v, out_hbm.at[pl.ds(base + s * _CH, _CH)])

    return k(table, idx)


_TN = 512
_NT = _N // _TN


def _vq_body(z_ref, iw_ref, v_ref, ct_ref, c4_ref,
             interp_ref, counts_ref, loss_ref, ms_ref, ow_acc):
    t = pl.program_id(0)
    j = pl.program_id(1)
    z = z_ref[0]
    ct = ct_ref[...]
    zc = jnp.dot(z, ct, preferred_element_type=jnp.float32)
    znorm = jnp.sum(z * z, axis=1, keepdims=True)
    cn = jnp.sum(ct * ct, axis=0, keepdims=True)
    d = znorm - 2.0 * zc + cn
    dmin = jnp.min(d, axis=1, keepdims=True)
    idx = jnp.argmin(d, axis=1)
    onehot = lax.broadcasted_iota(jnp.int32, (_TN, _K), 1) == idx[:, None]
    v = v_ref[...]
    wv = iw_ref[0] * v
    ow = jnp.where(onehot, wv, 0.0)

    first = jnp.logical_and(t == 0, j == 0)

    @pl.when(first)
    def _init():
        counts_ref[...] = jnp.zeros_like(counts_ref)
        loss_ref[...] = jnp.zeros_like(loss_ref)
        ms_ref[...] = jnp.zeros_like(ms_ref)

    @pl.when(j == 0)
    def _start_tile():
        ow_acc[...] = ow
        ms_ref[...] += jnp.sum(v).reshape(1, 1)

    @pl.when(j > 0)
    def _accum():
        ow_acc[...] += ow

    @pl.when(j == 7)
    def _emit():
        interp_ref[...] = jnp.dot(
            ow_acc[...], c4_ref[...], preferred_element_type=jnp.float32
        )

    ov = jnp.where(onehot, v, 0.0)
    counts_ref[...] += jnp.sum(ov, axis=0, keepdims=True)
    loss_ref[...] += jnp.sum(v * dmin).reshape(1, 1)


def _run_vq(zarr, iw_t, vf, ct, c4):
    return pl.pallas_call(
        _vq_body,
        grid=(_NT, 8),
        in_specs=[
            pl.BlockSpec((1, _TN, _D), lambda t, j: (j, t, 0)),
            pl.BlockSpec((1, _TN, 1), lambda t, j: (j, t, 0)),
            pl.BlockSpec((_TN, 1), lambda t, j: (t, 0)),
            pl.BlockSpec((_D, _K), lambda t, j: (0, 0)),
            pl.BlockSpec((_K, 4), lambda t, j: (0, 0)),
        ],
        out_specs=[
            pl.BlockSpec((_TN, 4), lambda t, j: (t, 0)),
            pl.BlockSpec((1, _K), lambda t, j: (0, 0)),
            pl.BlockSpec((1, 1), lambda t, j: (0, 0)),
            pl.BlockSpec((1, 1), lambda t, j: (0, 0)),
        ],
        out_shape=[
            jax.ShapeDtypeStruct((_N, 4), jnp.float32),
            jax.ShapeDtypeStruct((1, _K), jnp.float32),
            jax.ShapeDtypeStruct((1, 1), jnp.float32),
            jax.ShapeDtypeStruct((1, 1), jnp.float32),
        ],
        scratch_shapes=[pltpu.VMEM((_TN, _K), jnp.float32)],
    )(zarr, iw_t, vf, ct, c4)


def _sigm(x):
    return 1.0 / (1.0 + jnp.exp(-x))


def _fin_body(t4_ref, counts_ref, loss_ref, ms_ref,
              vql_ref, perp_ref, r0_ref, r1_ref, r2_ref):
    ms8 = ms_ref[...] * 8.0
    vql_ref[...] = (1.0 + _COMMIT) * loss_ref[...] / (ms8 * _D + 1e-9)
    avg = counts_ref[...] / (ms8 + 1e-9)
    perp_ref[...] = jnp.exp(-jnp.sum(avg * jnp.log(avg + 1e-10))).reshape(1, 1)

    trans = jnp.ones((1, _B), jnp.float32)
    a0 = jnp.zeros((1, _B), jnp.float32)
    a1 = jnp.zeros((1, _B), jnp.float32)
    a2 = jnp.zeros((1, _B), jnp.float32)
    for i in range(_NI):
        blk = t4_ref[i]
        sig = jnp.maximum(blk[0:1, :], 0.0)
        alpha = 1.0 - jnp.exp(-sig * float(_dt_np[i]))
        w = alpha * trans
        a0 = a0 + w * _sigm(blk[1:2, :])
        a1 = a1 + w * _sigm(blk[2:3, :])
        a2 = a2 + w * _sigm(blk[3:4, :])
        trans = trans * (1.0 - alpha + 1e-10)
    r0_ref[...] = a0
    r1_ref[...] = a1
    r2_ref[...] = a2


def _run_fin(t4, counts, loss, ms):
    return pl.pallas_call(
        _fin_body,
        out_shape=[
            jax.ShapeDtypeStruct((1, 1), jnp.float32),
            jax.ShapeDtypeStruct((1, 1), jnp.float32),
            jax.ShapeDtypeStruct((1, _B), jnp.float32),
            jax.ShapeDtypeStruct((1, _B), jnp.float32),
            jax.ShapeDtypeStruct((1, _B), jnp.float32),
        ],
    )(t4, counts, loss, ms)


def kernel(rays_o, rays_d, data_weight, codebook):
    tm = jnp.asarray(_tmid_np)
    pts = rays_o[:, None, :] + tm[None, :, None] * rays_d[:, None, :]
    valid = jnp.all((pts >= 0.0) & (pts < 1.0), axis=-1)
    flat_pts = pts.reshape(-1, 3)
    clipped = jnp.clip(flat_pts, 0.0, 1.0 - 1e-6)
    scaled = clipped * _R
    idx0f = jnp.clip(jnp.floor(scaled), 0.0, _R - 1)
    frac = scaled - idx0f
    idx0 = idx0f.astype(jnp.int32)
    offs = jnp.asarray(_offsets_np)
    corners = idx0[:, None, :] + offs[None, :, :]
    nids = (corners[..., 0] * (_R + 1) + corners[..., 1]) * (_R + 1) + corners[..., 2]
    w = jnp.where(offs[None, :, :] == 1, frac[:, None, :], 1.0 - frac[:, None, :])
    iweights = jnp.prod(w, axis=-1)

    nids_t = nids.T.reshape(-1)
    iw_t = iweights.T.reshape(8, _N, 1)
    vf = valid.reshape(-1, 1).astype(jnp.float32)

    zflat = _sc_gather(data_weight, nids_t)
    zarr = zflat.reshape(8, _N, _D)

    ct = codebook.T
    c4 = codebook[:, 0:4]

    interp4, counts, loss, ms = _run_vq(zarr, iw_t, vf, ct, c4)

    t4 = interp4.reshape(_B, _NI, 4).transpose(1, 2, 0)
    vql, perp, r0, r1, r2 = _run_fin(t4, counts, loss, ms)
    rgb = jnp.concatenate([r0, r1, r2], axis=0).T
    return (vql[0, 0], perp[0, 0], rgb)


# speedup vs baseline: 2.0628x; 1.2362x over previous
import functools

import numpy as np
import jax
import jax.numpy as jnp
from jax import lax
from jax.experimental import pallas as pl
from jax.experimental.pallas import tpu as pltpu
from jax.experimental.pallas import tpu_sc as plsc

_R = 64
_D = 32
_K = 512
_B = 1024
_NI = 16
_N = _B * _NI
_N8 = _N * 8
_COMMIT = 0.25

_t_np = np.linspace(0.05, 1.2, _NI + 1, dtype=np.float32)
_tmid_np = 0.5 * (_t_np[:-1] + _t_np[1:])
_dt_np = (_t_np[1:] - _t_np[:-1]).astype(np.float32)
_offsets_np = np.array(
    [[i, j, k] for i in (0, 1) for j in (0, 1) for k in (0, 1)], dtype=np.int32
)

_NW = 32
_PW = _N8 // _NW
_CH = 1024
_G = 128


def _sc_gather(table, idx):
    mesh = plsc.VectorSubcoreMesh(core_axis_name="c", subcore_axis_name="s")

    @functools.partial(
        pl.kernel,
        mesh=mesh,
        out_type=jax.ShapeDtypeStruct((_N8, _D), jnp.float32),
        scratch_types=[
            pltpu.VMEM((_PW,), jnp.int32),
            pltpu.VMEM((_CH, _D), jnp.float32),
            pltpu.SemaphoreType.DMA,
        ],
        compiler_params=pltpu.CompilerParams(use_tc_tiling_on_sc=False),
    )
    def k(table_hbm, idx_hbm, out_hbm, idx_v, rows_v, sem):
        wid = lax.axis_index("s") * 2 + lax.axis_index("c")
        base = wid * _PW
        pltpu.sync_copy(idx_hbm.at[pl.ds(base, _PW)], idx_v)
        for s in range(_PW // _CH):
            cps = [
                pltpu.async_copy(
                    table_hbm.at[idx_v.at[pl.ds(s * _CH + c * _G, _G)]],
                    rows_v.at[pl.ds(c * _G, _G)],
                    sem,
                )
                for c in range(_CH // _G)
            ]
            for cp in cps:
                cp.wait()
            pltpu.sync_copy(rows_v, out_hbm.at[pl.ds(base + s * _CH, _CH)])

    return k(table, idx)


_TN = 1024
_NT = _N // _TN


def _vq_body(z_ref, iw_ref, v_ref, cm2_ref, cn_ref, c4t_ref,
             interp_ref, counts_ref, loss_ref, ms_ref, ow_acc):
    t = pl.program_id(0)
    j = pl.program_id(1)
    z = z_ref[0]
    score = lax.dot_general(
        cm2_ref[...], z, (((1,), (1,)), ((), ())),
        preferred_element_type=jnp.float32,
    ) + cn_ref[...]
    idx = jnp.argmin(score, axis=0)
    onehot = lax.broadcasted_iota(jnp.int32, (_K, _TN), 0) == idx[None, :]
    v = v_ref[...]
    ov = jnp.where(onehot, v, 0.0)

    zz = jnp.sum(z * z, axis=1).reshape(1, _TN)
    part = jnp.sum(v * zz) + jnp.sum(ov * score)

    first = jnp.logical_and(t == 0, j == 0)

    @pl.when(first)
    def _init():
        counts_ref[...] = jnp.zeros_like(counts_ref)
        loss_ref[...] = jnp.zeros_like(loss_ref)
        ms_ref[...] = jnp.zeros_like(ms_ref)

    @pl.when(j == 0)
    def _mask_sum():
        ms_ref[...] += jnp.sum(v).reshape(1, 1)

    counts_ref[...] += jnp.sum(ov, axis=1, keepdims=True)
    loss_ref[...] += part.reshape(1, 1)

    wv = iw_ref[0] * v
    ow = jnp.where(onehot, wv, 0.0)

    @pl.when(j == 0)
    def _set():
        ow_acc[...] = ow

    @pl.when(j > 0)
    def _add():
        ow_acc[...] += ow

    @pl.when(j == 7)
    def _emit():
        interp_ref[...] = jnp.dot(
            c4t_ref[...], ow_acc[...], preferred_element_type=jnp.float32
        )


def _run_vq(zarr, iw_t, vf, cm2, cn, c4t):
    return pl.pallas_call(
        _vq_body,
        grid=(_NT, 8),
        in_specs=[
            pl.BlockSpec((1, _TN, _D), lambda t, j: (j, t, 0)),
            pl.BlockSpec((1, 1, _TN), lambda t, j: (j, 0, t)),
            pl.BlockSpec((1, _TN), lambda t, j: (0, t)),
            pl.BlockSpec((_K, _D), lambda t, j: (0, 0)),
            pl.BlockSpec((_K, 1), lambda t, j: (0, 0)),
            pl.BlockSpec((4, _K), lambda t, j: (0, 0)),
        ],
        out_specs=[
            pl.BlockSpec((4, _TN), lambda t, j: (0, t)),
            pl.BlockSpec((_K, 1), lambda t, j: (0, 0)),
            pl.BlockSpec((1, 1), lambda t, j: (0, 0)),
            pl.BlockSpec((1, 1), lambda t, j: (0, 0)),
        ],
        out_shape=[
            jax.ShapeDtypeStruct((4, _N), jnp.float32),
            jax.ShapeDtypeStruct((_K, 1), jnp.float32),
            jax.ShapeDtypeStruct((1, 1), jnp.float32),
            jax.ShapeDtypeStruct((1, 1), jnp.float32),
        ],
        scratch_shapes=[pltpu.VMEM((_K, _TN), jnp.float32)],
    )(zarr, iw_t, vf, cm2, cn, c4t)


def _sigm(x):
    return 1.0 / (1.0 + jnp.exp(-x))


def _fin_body(t4_ref, counts_ref, loss_ref, ms_ref,
              vql_ref, perp_ref, r0_ref, r1_ref, r2_ref):
    ms8 = ms_ref[...] * 8.0
    vql_ref[...] = (1.0 + _COMMIT) * loss_ref[...] / (ms8 * _D + 1e-9)
    avg = counts_ref[...] / (ms8 + 1e-9)
    perp_ref[...] = jnp.exp(-jnp.sum(avg * jnp.log(avg + 1e-10))).reshape(1, 1)

    trans = jnp.ones((1, _B), jnp.float32)
    a0 = jnp.zeros((1, _B), jnp.float32)
    a1 = jnp.zeros((1, _B), jnp.float32)
    a2 = jnp.zeros((1, _B), jnp.float32)
    for i in range(_NI):
        blk = t4_ref[i]
        sig = jnp.maximum(blk[0:1, :], 0.0)
        alpha = 1.0 - jnp.exp(-sig * float(_dt_np[i]))
        w = alpha * trans
        a0 = a0 + w * _sigm(blk[1:2, :])
        a1 = a1 + w * _sigm(blk[2:3, :])
        a2 = a2 + w * _sigm(blk[3:4, :])
        trans = trans * (1.0 - alpha + 1e-10)
    r0_ref[...] = a0
    r1_ref[...] = a1
    r2_ref[...] = a2


def _run_fin(t4, counts, loss, ms):
    return pl.pallas_call(
        _fin_body,
        out_shape=[
            jax.ShapeDtypeStruct((1, 1), jnp.float32),
            jax.ShapeDtypeStruct((1, 1), jnp.float32),
            jax.ShapeDtypeStruct((1, _B), jnp.float32),
            jax.ShapeDtypeStruct((1, _B), jnp.float32),
            jax.ShapeDtypeStruct((1, _B), jnp.float32),
        ],
    )(t4, counts, loss, ms)


def _prep(rays_o, rays_d):
    tm = jnp.asarray(_tmid_np)
    pts = rays_o[:, None, :] + tm[None, :, None] * rays_d[:, None, :]
    valid = jnp.all((pts >= 0.0) & (pts < 1.0), axis=-1)
    flat_pts = pts.reshape(-1, 3)
    clipped = jnp.clip(flat_pts, 0.0, 1.0 - 1e-6)
    scaled = clipped * _R
    idx0f = jnp.clip(jnp.floor(scaled), 0.0, _R - 1)
    frac = scaled - idx0f
    idx0 = idx0f.astype(jnp.int32)
    offs = jnp.asarray(_offsets_np)
    corners = idx0[:, None, :] + offs[None, :, :]
    nids = (corners[..., 0] * (_R + 1) + corners[..., 1]) * (_R + 1) + corners[..., 2]
    w = jnp.where(offs[None, :, :] == 1, frac[:, None, :], 1.0 - frac[:, None, :])
    iweights = jnp.prod(w, axis=-1)
    return nids, iweights, valid


def kernel(rays_o, rays_d, data_weight, codebook):
    nids, iweights, valid = _prep(rays_o, rays_d)

    nids_t = nids.T.reshape(-1)
    iw_t = iweights.T.reshape(8, 1, _N)
    vf = valid.reshape(1, -1).astype(jnp.float32)

    zflat = _sc_gather(data_weight, nids_t)
    zarr = zflat.reshape(8, _N, _D)

    cm2 = -2.0 * codebook
    cn = jnp.sum(codebook * codebook, axis=1).reshape(_K, 1)
    c4t = codebook[:, 0:4].T

    interp_t, counts, loss, ms = _run_vq(zarr, iw_t, vf, cm2, cn, c4t)

    t4 = interp_t.reshape(4, _B, _NI).transpose(2, 0, 1)
    vql, perp, r0, r1, r2 = _run_fin(t4, counts, loss, ms)
    rgb = jnp.concatenate([r0, r1, r2], axis=0).T
    return (vql[0, 0], perp[0, 0], rgb)


# speedup vs baseline: 2.7179x; 1.3176x over previous
import functools

import numpy as np
import jax
import jax.numpy as jnp
from jax import lax
from jax.experimental import pallas as pl
from jax.experimental.pallas import tpu as pltpu
from jax.experimental.pallas import tpu_sc as plsc

_R = 64
_D = 32
_K = 512
_B = 1024
_NI = 16
_N = _B * _NI
_N8 = _N * 8
_COMMIT = 0.25

_t_np = np.linspace(0.05, 1.2, _NI + 1, dtype=np.float32)
_tmid_np = 0.5 * (_t_np[:-1] + _t_np[1:])
_dt_np = (_t_np[1:] - _t_np[:-1]).astype(np.float32)
_offsets_np = np.array(
    [[i, j, k] for i in (0, 1) for j in (0, 1) for k in (0, 1)], dtype=np.int32
)

_NW = 32

_PW = _N8 // _NW
_CH = 1024
_G = 128


def _sc_gather(table, idx):
    mesh = plsc.VectorSubcoreMesh(core_axis_name="c", subcore_axis_name="s")

    @functools.partial(
        pl.kernel,
        mesh=mesh,
        out_type=jax.ShapeDtypeStruct((_N8, _D), jnp.float32),
        scratch_types=[
            pltpu.VMEM((_PW,), jnp.int32),
            pltpu.VMEM((_CH, _D), jnp.float32),
            pltpu.SemaphoreType.DMA,
        ],
        compiler_params=pltpu.CompilerParams(use_tc_tiling_on_sc=False),
    )
    def k(table_hbm, idx_hbm, out_hbm, idx_v, rows_v, sem):
        wid = lax.axis_index("s") * 2 + lax.axis_index("c")
        base = wid * _PW
        pltpu.sync_copy(idx_hbm.at[pl.ds(base, _PW)], idx_v)
        for s in range(_PW // _CH):
            cps = [
                pltpu.async_copy(
                    table_hbm.at[idx_v.at[pl.ds(s * _CH + c * _G, _G)]],
                    rows_v.at[pl.ds(c * _G, _G)],
                    sem,
                )
                for c in range(_CH // _G)
            ]
            for cp in cps:
                cp.wait()
            pltpu.sync_copy(rows_v, out_hbm.at[pl.ds(base + s * _CH, _CH)])

    return k(table, idx)


_TN = 1024
_NT8 = _N8 // _TN


def _vq_body(z_ref, vr_ref, vc_ref, cm2_ref, cn_ref,
             idx_ref, loss_ref, ms_ref):
    t = pl.program_id(0)
    z = z_ref[...]
    score = lax.dot_general(
        cm2_ref[...], z, (((1,), (1,)), ((), ())),
        preferred_element_type=jnp.float32,
    ) + cn_ref[...]
    mn = jnp.min(score, axis=0).reshape(1, _TN)
    idx = jnp.argmin(score, axis=0)
    idx_ref[...] = idx.reshape(1, 1, _TN)

    zz = jnp.sum(z * z, axis=1, keepdims=True)
    part = jnp.sum(vc_ref[...] * zz) + jnp.sum(vr_ref[...] * mn)

    @pl.when(t == 0)
    def _init():
        loss_ref[...] = jnp.zeros_like(loss_ref)
        ms_ref[...] = jnp.zeros_like(ms_ref)

    loss_ref[...] += part.reshape(1, 1)
    ms_ref[...] += jnp.sum(vr_ref[...]).reshape(1, 1)


def _run_vq(zflat, vrow8, vcol8, cm2, cn):
    return pl.pallas_call(
        _vq_body,
        grid=(_NT8,),
        in_specs=[
            pl.BlockSpec((_TN, _D), lambda t: (t, 0)),
            pl.BlockSpec((1, _TN), lambda t: (0, t)),
            pl.BlockSpec((_TN, 1), lambda t: (t, 0)),
            pl.BlockSpec((_K, _D), lambda t: (0, 0)),
            pl.BlockSpec((_K, 1), lambda t: (0, 0)),
        ],
        out_specs=[
            pl.BlockSpec((1, 1, _TN), lambda t: (t, 0, 0)),
            pl.BlockSpec((1, 1), lambda t: (0, 0)),
            pl.BlockSpec((1, 1), lambda t: (0, 0)),
        ],
        out_shape=[
            jax.ShapeDtypeStruct((_NT8, 1, _TN), jnp.int32),
            jax.ShapeDtypeStruct((1, 1), jnp.float32),
            jax.ShapeDtypeStruct((1, 1), jnp.float32),
        ],
    )(zflat, vrow8, vcol8, cm2, cn)


_PW2 = _N // _NW
_L = 16


def _sc_interp(idx8, wv8, vmask, c4t):
    mesh = plsc.VectorSubcoreMesh(core_axis_name="c", subcore_axis_name="s")

    @functools.partial(
        pl.kernel,
        mesh=mesh,
        out_type=[
            jax.ShapeDtypeStruct((4, _N), jnp.float32),
            jax.ShapeDtypeStruct((_NW, _K), jnp.float32),
        ],
        scratch_types=[
            pltpu.VMEM((4, _K), jnp.float32),
            pltpu.VMEM((8, _PW2), jnp.int32),
            pltpu.VMEM((8, _PW2), jnp.float32),
            pltpu.VMEM((_PW2,), jnp.float32),
            pltpu.VMEM((4, _PW2), jnp.float32),
            pltpu.VMEM((_K,), jnp.float32),
        ],
        compiler_params=pltpu.CompilerParams(
            use_tc_tiling_on_sc=False, needs_layout_passes=False),
    )
    def k(idx_hbm, wv_hbm, v_hbm, c4_hbm, interp_hbm, cnt_hbm,
          c4_v, idx_v, wv_v, v_v, out_v, cnt_v):
        wid = lax.axis_index("s") * 2 + lax.axis_index("c")
        base = wid * _PW2
        pltpu.sync_copy(c4_hbm, c4_v)
        pltpu.sync_copy(idx_hbm.at[:, pl.ds(base, _PW2)], idx_v)
        pltpu.sync_copy(wv_hbm.at[:, pl.ds(base, _PW2)], wv_v)
        pltpu.sync_copy(v_hbm.at[pl.ds(base, _PW2)], v_v)
        for i in range(_K // _L):
            cnt_v[pl.ds(i * _L, _L)] = jnp.zeros((_L,), jnp.float32)

        def body(g, carry):
            s = pl.ds(g * _L, _L)
            vg = v_v[s]
            accs = [jnp.zeros((_L,), jnp.float32) for _ in range(4)]
            for j in range(8):
                ixg = idx_v[j, s]
                wg = wv_v[j, s]
                for c in range(4):
                    accs[c] = accs[c] + wg * plsc.load_gather(c4_v, [
                        jnp.full((_L,), c, jnp.int32), ixg])
                plsc.addupdate_scatter(cnt_v, [ixg], vg)
            for c in range(4):
                out_v[c, s] = accs[c]
            return carry

        lax.fori_loop(0, _PW2 // _L, body, 0)
        pltpu.sync_copy(out_v, interp_hbm.at[:, pl.ds(base, _PW2)])
        pltpu.sync_copy(cnt_v, cnt_hbm.at[wid])

    return k(idx8, wv8, vmask, c4t)


def _sigm(x):
    return 1.0 / (1.0 + jnp.exp(-x))


def _fin_body(t4_ref, cparts_ref, loss_ref, ms_ref,
              vql_ref, perp_ref, r0_ref, r1_ref, r2_ref):
    ms8 = ms_ref[...]
    vql_ref[...] = (1.0 + _COMMIT) * loss_ref[...] / (ms8 * _D + 1e-9)
    counts = jnp.sum(cparts_ref[...], axis=0, keepdims=True)
    avg = counts / (ms8 + 1e-9)
    perp_ref[...] = jnp.exp(-jnp.sum(avg * jnp.log(avg + 1e-10))).reshape(1, 1)

    trans = jnp.ones((1, _B), jnp.float32)
    a0 = jnp.zeros((1, _B), jnp.float32)
    a1 = jnp.zeros((1, _B), jnp.float32)
    a2 = jnp.zeros((1, _B), jnp.float32)
    for i in range(_NI):
        blk = t4_ref[i]
        sig = jnp.maximum(blk[0:1, :], 0.0)
        alpha = 1.0 - jnp.exp(-sig * float(_dt_np[i]))
        w = alpha * trans
        a0 = a0 + w * _sigm(blk[1:2, :])
        a1 = a1 + w * _sigm(blk[2:3, :])
        a2 = a2 + w * _sigm(blk[3:4, :])
        trans = trans * (1.0 - alpha + 1e-10)
    r0_ref[...] = a0
    r1_ref[...] = a1
    r2_ref[...] = a2


def _run_fin(t4, cparts, loss, ms):
    return pl.pallas_call(
        _fin_body,
        out_shape=[
            jax.ShapeDtypeStruct((1, 1), jnp.float32),
            jax.ShapeDtypeStruct((1, 1), jnp.float32),
            jax.ShapeDtypeStruct((1, _B), jnp.float32),
            jax.ShapeDtypeStruct((1, _B), jnp.float32),
            jax.ShapeDtypeStruct((1, _B), jnp.float32),
        ],
    )(t4, cparts, loss, ms)


def _prep(rays_o, rays_d):
    tm = jnp.asarray(_tmid_np)
    pts = rays_o[:, None, :] + tm[None, :, None] * rays_d[:, None, :]
    valid = jnp.all((pts >= 0.0) & (pts < 1.0), axis=-1)
    flat_pts = pts.reshape(-1, 3)
    clipped = jnp.clip(flat_pts, 0.0, 1.0 - 1e-6)
    scaled = clipped * _R
    idx0f = jnp.clip(jnp.floor(scaled), 0.0, _R - 1)
    frac = scaled - idx0f
    idx0 = idx0f.astype(jnp.int32)
    offs = jnp.asarray(_offsets_np)
    corners = idx0[:, None, :] + offs[None, :, :]
    nids = (corners[..., 0] * (_R + 1) + corners[..., 1]) * (_R + 1) + corners[..., 2]
    w = jnp.where(offs[None, :, :] == 1, frac[:, None, :], 1.0 - frac[:, None, :])
    iweights = jnp.prod(w, axis=-1)
    return nids, iweights, valid


def kernel(rays_o, rays_d, data_weight, codebook):
    nids, iweights, valid = _prep(rays_o, rays_d)

    nids_t = nids.T.reshape(-1)
    vf = valid.reshape(-1).astype(jnp.float32)
    vrow8 = jnp.tile(vf, 8).reshape(1, _N8)
    vcol8 = vrow8.reshape(_N8, 1)
    wv8 = (iweights * vf[:, None]).T

    zflat = _sc_gather(data_weight, nids_t)

    cm2 = -2.0 * codebook
    cn = jnp.sum(codebook * codebook, axis=1).reshape(_K, 1)
    c4t = codebook[:, 0:4].T

    idx_out, loss, ms = _run_vq(zflat, vrow8, vcol8, cm2, cn)
    idx8 = idx_out.reshape(8, _N)

    interp_t, cparts = _sc_interp(idx8, wv8, vf, c4t)

    t4 = interp_t.reshape(4, _B, _NI).transpose(2, 0, 1)
    vql, perp, r0, r1, r2 = _run_fin(t4, cparts, loss, ms)
    rgb = jnp.concatenate([r0, r1, r2], axis=0).T
    return (vql[0, 0], perp[0, 0], rgb)


# speedup vs baseline: 3.0093x; 1.1072x over previous
import functools

import numpy as np
import jax
import jax.numpy as jnp
from jax import lax
from jax.experimental import pallas as pl
from jax.experimental.pallas import tpu as pltpu
from jax.experimental.pallas import tpu_sc as plsc

_R = 64
_D = 32
_K = 512
_B = 1024
_NI = 16
_N = _B * _NI
_N8 = _N * 8
_COMMIT = 0.25

_t_np = np.linspace(0.05, 1.2, _NI + 1, dtype=np.float32)
_tmid_np = 0.5 * (_t_np[:-1] + _t_np[1:])
_dt_np = (_t_np[1:] - _t_np[:-1]).astype(np.float32)
_offsets_np = np.array(
    [[i, j, k] for i in (0, 1) for j in (0, 1) for k in (0, 1)], dtype=np.int32
)

_NW = 32

_PW = _N8 // _NW
_CH = 1024
_G = 128


def _sc_gather(table, idx):
    mesh = plsc.VectorSubcoreMesh(core_axis_name="c", subcore_axis_name="s")

    @functools.partial(
        pl.kernel,
        mesh=mesh,
        out_type=jax.ShapeDtypeStruct((_N8, _D), jnp.float32),
        scratch_types=[
            pltpu.VMEM((_PW,), jnp.int32),
            pltpu.VMEM((_CH, _D), jnp.float32),
            pltpu.SemaphoreType.DMA,
        ],
        compiler_params=pltpu.CompilerParams(use_tc_tiling_on_sc=False),
    )
    def k(table_hbm, idx_hbm, out_hbm, idx_v, rows_v, sem):
        wid = lax.axis_index("s") * 2 + lax.axis_index("c")
        base = wid * _PW
        pltpu.sync_copy(idx_hbm.at[pl.ds(base, _PW)], idx_v)
        for s in range(_PW // _CH):
            cps = [
                pltpu.async_copy(
                    table_hbm.at[idx_v.at[pl.ds(s * _CH + c * _G, _G)]],
                    rows_v.at[pl.ds(c * _G, _G)],
                    sem,
                )
                for c in range(_CH // _G)
            ]
            for cp in cps:
                cp.wait()
            pltpu.sync_copy(rows_v, out_hbm.at[pl.ds(base + s * _CH, _CH)])

    return k(table, idx)


_TN = 1024
_NT8 = _N8 // _TN


def _vq_body(z_ref, vr_ref, cm2_ref, cn_ref,
             idx_ref, loss_ref, ms_ref):
    t = pl.program_id(0)
    z = z_ref[...]
    score = lax.dot_general(
        cm2_ref[...], z, (((1,), (1,)), ((), ())),
        preferred_element_type=jnp.float32,
    ) + cn_ref[...]
    mn = jnp.min(score, axis=0).reshape(1, _TN)
    idx = jnp.argmin(score, axis=0)
    idx_ref[...] = idx.reshape(1, 1, _TN)

    vr = vr_ref[...]
    zz = jnp.sum(z * z, axis=1, keepdims=True)
    part = lax.dot_general(
        vr, zz, (((1,), (0,)), ((), ())),
        preferred_element_type=jnp.float32,
    ) + jnp.sum(vr * mn).reshape(1, 1)

    @pl.when(t == 0)
    def _init():
        loss_ref[...] = jnp.zeros_like(loss_ref)
        ms_ref[...] = jnp.zeros_like(ms_ref)

    loss_ref[...] += part
    ms_ref[...] += jnp.sum(vr).reshape(1, 1)


def _run_vq(zflat, vrow8, cm2, cn):
    return pl.pallas_call(
        _vq_body,
        grid=(_NT8,),
        in_specs=[
            pl.BlockSpec((_TN, _D), lambda t: (t, 0)),
            pl.BlockSpec((1, _TN), lambda t: (0, t)),
            pl.BlockSpec((_K, _D), lambda t: (0, 0)),
            pl.BlockSpec((_K, 1), lambda t: (0, 0)),
        ],
        out_specs=[
            pl.BlockSpec((1, 1, _TN), lambda t: (t, 0, 0)),
            pl.BlockSpec((1, 1), lambda t: (0, 0)),
            pl.BlockSpec((1, 1), lambda t: (0, 0)),
        ],
        out_shape=[
            jax.ShapeDtypeStruct((_NT8, 1, _TN), jnp.int32),
            jax.ShapeDtypeStruct((1, 1), jnp.float32),
            jax.ShapeDtypeStruct((1, 1), jnp.float32),
        ],
    )(zflat, vrow8, cm2, cn)


_PW2 = _N // _NW
_L = 16


def _sc_interp(idx8, wv8, vmask, c4t):
    mesh = plsc.VectorSubcoreMesh(core_axis_name="c", subcore_axis_name="s")

    @functools.partial(
        pl.kernel,
        mesh=mesh,
        out_type=[
            jax.ShapeDtypeStruct((4, _N), jnp.float32),
            jax.ShapeDtypeStruct((_NW, _K), jnp.float32),
        ],
        scratch_types=[
            pltpu.VMEM((4, _K), jnp.float32),
            pltpu.VMEM((8, _PW2), jnp.int32),
            pltpu.VMEM((8, _PW2), jnp.float32),
            pltpu.VMEM((_PW2,), jnp.float32),
            pltpu.VMEM((4, _PW2), jnp.float32),
            pltpu.VMEM((_K,), jnp.float32),
        ],
        compiler_params=pltpu.CompilerParams(
            use_tc_tiling_on_sc=False, needs_layout_passes=False),
    )
    def k(idx_hbm, wv_hbm, v_hbm, c4_hbm, interp_hbm, cnt_hbm,
          c4_v, idx_v, wv_v, v_v, out_v, cnt_v):
        wid = lax.axis_index("s") * 2 + lax.axis_index("c")
        base = wid * _PW2
        pltpu.sync_copy(c4_hbm, c4_v)
        pltpu.sync_copy(idx_hbm.at[:, pl.ds(base, _PW2)], idx_v)
        pltpu.sync_copy(wv_hbm.at[:, pl.ds(base, _PW2)], wv_v)
        pltpu.sync_copy(v_hbm.at[pl.ds(base, _PW2)], v_v)
        for i in range(_K // _L):
            cnt_v[pl.ds(i * _L, _L)] = jnp.zeros((_L,), jnp.float32)

        def body(g, carry):
            s = pl.ds(g * _L, _L)
            vg = v_v[s]
            accs = [jnp.zeros((_L,), jnp.float32) for _ in range(4)]
            for j in range(8):
                ixg = idx_v[j, s]
                wg = wv_v[j, s]
                for c in range(4):
                    accs[c] = accs[c] + wg * plsc.load_gather(c4_v, [
                        jnp.full((_L,), c, jnp.int32), ixg])
                plsc.addupdate_scatter(cnt_v, [ixg], vg)
            for c in range(4):
                out_v[c, s] = accs[c]
            return carry

        lax.fori_loop(0, _PW2 // _L, body, 0)
        pltpu.sync_copy(out_v, interp_hbm.at[:, pl.ds(base, _PW2)])
        pltpu.sync_copy(cnt_v, cnt_hbm.at[wid])

    return k(idx8, wv8, vmask, c4t)


def _sigm(x):
    return 1.0 / (1.0 + jnp.exp(-x))


def _fin_body(t4_ref, cparts_ref, loss_ref, ms_ref,
              vql_ref, perp_ref, r0_ref, r1_ref, r2_ref):
    ms8 = ms_ref[...]
    vql_ref[...] = (1.0 + _COMMIT) * loss_ref[...] / (ms8 * _D + 1e-9)
    counts = jnp.sum(cparts_ref[...], axis=0, keepdims=True)
    avg = counts / (ms8 + 1e-9)
    perp_ref[...] = jnp.exp(-jnp.sum(avg * jnp.log(avg + 1e-10))).reshape(1, 1)

    trans = jnp.ones((1, _B), jnp.float32)
    a0 = jnp.zeros((1, _B), jnp.float32)
    a1 = jnp.zeros((1, _B), jnp.float32)
    a2 = jnp.zeros((1, _B), jnp.float32)
    for i in range(_NI):
        blk = t4_ref[i]
        sig = jnp.maximum(blk[0:1, :], 0.0)
        alpha = 1.0 - jnp.exp(-sig * float(_dt_np[i]))
        w = alpha * trans
        a0 = a0 + w * _sigm(blk[1:2, :])
        a1 = a1 + w * _sigm(blk[2:3, :])
        a2 = a2 + w * _sigm(blk[3:4, :])
        trans = trans * (1.0 - alpha + 1e-10)
    r0_ref[...] = a0
    r1_ref[...] = a1
    r2_ref[...] = a2


def _run_fin(t4, cparts, loss, ms):
    return pl.pallas_call(
        _fin_body,
        out_shape=[
            jax.ShapeDtypeStruct((1, 1), jnp.float32),
            jax.ShapeDtypeStruct((1, 1), jnp.float32),
            jax.ShapeDtypeStruct((1, _B), jnp.float32),
            jax.ShapeDtypeStruct((1, _B), jnp.float32),
            jax.ShapeDtypeStruct((1, _B), jnp.float32),
        ],
    )(t4, cparts, loss, ms)


def _prep(rays_o, rays_d):
    tm = jnp.asarray(_tmid_np)
    pts = rays_o[:, None, :] + tm[None, :, None] * rays_d[:, None, :]
    valid = jnp.all((pts >= 0.0) & (pts < 1.0), axis=-1)
    flat_pts = pts.reshape(-1, 3)
    clipped = jnp.clip(flat_pts, 0.0, 1.0 - 1e-6)
    scaled = clipped * _R
    idx0f = jnp.clip(jnp.floor(scaled), 0.0, _R - 1)
    frac = scaled - idx0f
    idx0 = idx0f.astype(jnp.int32)
    offs = jnp.asarray(_offsets_np)
    corners = idx0[None, :, :] + offs[:, None, :]
    nids8 = (corners[..., 0] * (_R + 1) + corners[..., 1]) * (_R + 1) + corners[..., 2]
    w = jnp.where(offs[:, None, :] == 1, frac[None, :, :], 1.0 - frac[None, :, :])
    iw8 = jnp.prod(w, axis=-1)
    return nids8, iw8, valid


def kernel(rays_o, rays_d, data_weight, codebook):
    nids8, iw8, valid = _prep(rays_o, rays_d)

    nids_t = nids8.reshape(-1)
    vf = valid.reshape(-1).astype(jnp.float32)
    vrow8 = jnp.tile(vf, 8).reshape(1, _N8)
    wv8 = iw8 * vf[None, :]

    zflat = _sc_gather(data_weight, nids_t)

    cm2 = -2.0 * codebook
    cn = jnp.sum(codebook * codebook, axis=1).reshape(_K, 1)
    c4t = codebook[:, 0:4].T

    idx_out, loss, ms = _run_vq(zflat, vrow8, cm2, cn)
    idx8 = idx_out.reshape(8, _N)

    interp_t, cparts = _sc_interp(idx8, wv8, vf, c4t)

    t4 = interp_t.reshape(4, _B, _NI).transpose(2, 0, 1)
    vql, perp, r0, r1, r2 = _run_fin(t4, cparts, loss, ms)
    rgb = jnp.concatenate([r0, r1, r2], axis=0).T
    return (vql[0, 0], perp[0, 0], rgb)
